# baseline (speedup 1.0000x reference)
import jax
import jax.numpy as jnp
from jax import lax
from jax.experimental import pallas as pl
from jax.experimental.pallas import tpu as pltpu
from jax.experimental.pallas import tpu_sc as plsc

N_NODES = 10000
D = 128
N_CORES = 2
N_SUB = 16
N_WORK = N_CORES * N_SUB
CHUNK = 128
ROWS_PER_TILE = 624
TAIL_BASE = ROWS_PER_TILE * N_SUB
TAIL_ROWS = N_NODES - TAIL_BASE
TRASH_ROWS = 1024
ACC_ROWS = N_NODES + TRASH_ROWS


def _sc_agg_body(feats, srcs, dsts, zeros, out, idx_s, idx_d, rows, acc, sem):
    c = lax.axis_index("c")
    s = lax.axis_index("s")
    w = c * N_SUB + s
    rpw = idx_s.shape[0]

    base = s * ROWS_PER_TILE

    def init_from(ref):
        pltpu.sync_copy(ref.at[pl.ds(base, ROWS_PER_TILE)],
                        acc.at[pl.ds(base, ROWS_PER_TILE)])

        @pl.when(s == N_SUB - 1)
        def _():
            pltpu.sync_copy(ref.at[pl.ds(TAIL_BASE, TAIL_ROWS)],
                            acc.at[pl.ds(TAIL_BASE, TAIL_ROWS)])

    @pl.when(c == 1)
    def _():
        init_from(feats)

    @pl.when(c != 1)
    def _():
        init_from(zeros)

    pltpu.sync_copy(srcs.at[pl.ds(w * rpw, rpw)], idx_s)
    pltpu.sync_copy(dsts.at[pl.ds(w * rpw, rpw)], idx_d)
    plsc.subcore_barrier()

    def step(k, carry):
        pltpu.async_copy(feats.at[idx_s.at[k]], rows, sem).wait()
        pltpu.sync_copy(rows, acc.at[idx_d.at[k]], add=True)
        return carry

    lax.fori_loop(0, rpw, step, 0)

    plsc.subcore_barrier()
    pltpu.sync_copy(acc.at[pl.ds(base, ROWS_PER_TILE)],
                    out.at[c, pl.ds(base, ROWS_PER_TILE)])

    @pl.when(s == N_SUB - 1)
    def _():
        pltpu.sync_copy(acc.at[pl.ds(TAIL_BASE, TAIL_ROWS)],
                        out.at[c, pl.ds(TAIL_BASE, TAIL_ROWS)])


def _sc_agg(feats, srcs, dsts, zeros):
    rpw = srcs.shape[0] // N_WORK
    fn = pl.kernel(
        _sc_agg_body,
        out_type=jax.ShapeDtypeStruct((N_CORES, N_NODES, D), jnp.float32),
        mesh=plsc.VectorSubcoreMesh(core_axis_name="c", subcore_axis_name="s",
                                    num_cores=N_CORES, num_subcores=N_SUB),
        scratch_types=[
            pltpu.VMEM((rpw, CHUNK), jnp.int32),
            pltpu.VMEM((rpw, CHUNK), jnp.int32),
            pltpu.VMEM((CHUNK, D), jnp.float32),
            pltpu.VMEM_SHARED((ACC_ROWS, D), jnp.float32),
            pltpu.SemaphoreType.DMA,
        ],
    )
    return fn(feats, srcs, dsts, zeros)


ROW_BLK = 2000


def _mlp_body(p_ref, wa, ba, wb, bb, out_ref):
    h = p_ref[0] + p_ref[1]
    t = jnp.maximum(jnp.dot(h, wa[...], preferred_element_type=jnp.float32) + ba[...], 0.0)
    out_ref[...] = jnp.dot(t, wb[...], preferred_element_type=jnp.float32) + bb[...]


def _mlp(p, Wa, ba, Wb, bb):
    return pl.pallas_call(
        _mlp_body,
        grid=(N_NODES // ROW_BLK,),
        in_specs=[
            pl.BlockSpec((N_CORES, ROW_BLK, D), lambda i: (0, i, 0)),
            pl.BlockSpec((D, D), lambda i: (0, 0)),
            pl.BlockSpec((1, D), lambda i: (0, 0)),
            pl.BlockSpec((D, D), lambda i: (0, 0)),
            pl.BlockSpec((1, D), lambda i: (0, 0)),
        ],
        out_specs=pl.BlockSpec((ROW_BLK, D), lambda i: (i, 0)),
        out_shape=jax.ShapeDtypeStruct((N_NODES, D), jnp.float32),
    )(p, Wa, ba.reshape(1, D), Wb, bb.reshape(1, D))


def _mlp_pool_body(p_ref, wa, ba, wb, bb, wc, bcp, out_ref, acc):
    i = pl.program_id(0)

    @pl.when(i == 0)
    def _():
        acc[...] = jnp.zeros_like(acc)

    h = p_ref[0] + p_ref[1]
    t = jnp.maximum(jnp.dot(h, wa[...], preferred_element_type=jnp.float32) + ba[...], 0.0)
    h2 = jnp.dot(t, wb[...], preferred_element_type=jnp.float32) + bb[...]
    acc[...] += jnp.sum(h2, axis=0, keepdims=True)

    @pl.when(i == pl.num_programs(0) - 1)
    def _():
        out_ref[...] = jnp.dot(acc[...] * (1.0 / N_NODES), wc[...],
                               preferred_element_type=jnp.float32) + bcp[...]


def _mlp_pool(p, Wa, ba, Wb, bb, Wcp, bcp):
    return pl.pallas_call(
        _mlp_pool_body,
        grid=(N_NODES // ROW_BLK,),
        in_specs=[
            pl.BlockSpec((N_CORES, ROW_BLK, D), lambda i: (0, i, 0)),
            pl.BlockSpec((D, D), lambda i: (0, 0)),
            pl.BlockSpec((1, D), lambda i: (0, 0)),
            pl.BlockSpec((D, D), lambda i: (0, 0)),
            pl.BlockSpec((1, D), lambda i: (0, 0)),
            pl.BlockSpec((D, D), lambda i: (0, 0)),
            pl.BlockSpec((1, D), lambda i: (0, 0)),
        ],
        out_specs=pl.BlockSpec((1, D), lambda i: (0, 0)),
        out_shape=jax.ShapeDtypeStruct((1, D), jnp.float32),
        scratch_shapes=[pltpu.VMEM((1, D), jnp.float32)],
    )(p, Wa, ba.reshape(1, D), Wb, bb.reshape(1, D), Wcp, bcp)


def kernel(x, edge_index, W1a, b1a, W1b, b1b, W2a, b2a, W2b, b2b, Wc, bc):
    ei = edge_index.astype(jnp.int32)
    src, dst = ei[0], ei[1]
    n_edges = src.shape[0]
    rpw = -(-n_edges // (N_WORK * CHUNK))
    rpw = -(-rpw // 8) * 8
    pad = N_WORK * rpw * CHUNK - n_edges
    src_p = jnp.concatenate([src, jnp.zeros((pad,), jnp.int32)]).reshape(N_WORK * rpw, CHUNK)
    trash = N_NODES + jnp.arange(pad, dtype=jnp.int32) % TRASH_ROWS
    dst_p = jnp.concatenate([dst, trash]).reshape(N_WORK * rpw, CHUNK)
    zeros = jnp.zeros((N_NODES, D), jnp.float32)

    p1 = _sc_agg(x, src_p, dst_p, zeros)
    h1 = _mlp(p1, W1a, b1a, W1b, b1b)
    p2 = _sc_agg(h1, src_p, dst_p, zeros)

    n_cls = Wc.shape[1]
    Wcp = jnp.pad(Wc, ((0, 0), (0, D - n_cls)))
    bcp = jnp.pad(bc, (0, D - n_cls)).reshape(1, D)
    out = _mlp_pool(p2, W2a, b2a, W2b, b2b, Wcp, bcp)
    return out[:, :n_cls]


# speedup vs baseline: 1.0780x; 1.0780x over previous
import jax
import jax.numpy as jnp
from jax import lax
from jax.experimental import pallas as pl
from jax.experimental.pallas import tpu as pltpu
from jax.experimental.pallas import tpu_sc as plsc

N_NODES = 10000
D = 128
N_CORES = 2
N_SUB = 16
N_WORK = N_CORES * N_SUB
CHUNK = 128
ROWS_PER_TILE = 624
TAIL_BASE = ROWS_PER_TILE * N_SUB
TAIL_ROWS = N_NODES - TAIL_BASE
TRASH_ROWS = 912
ACC_ROWS = N_NODES + TRASH_ROWS


def _sc_agg_body(feats, srcs, dsts, zeros, out, idx_s, idx_d0, idx_d1,
                 rows0, rows1, acc, gsem0, gsem1, ssem0, ssem1, dsem0, dsem1):
    c = lax.axis_index("c")
    s = lax.axis_index("s")
    w = c * N_SUB + s
    rpw = idx_s.shape[0]

    base = s * ROWS_PER_TILE

    def init_from(ref):
        pltpu.sync_copy(ref.at[pl.ds(base, ROWS_PER_TILE)],
                        acc.at[pl.ds(base, ROWS_PER_TILE)])

        @pl.when(s == N_SUB - 1)
        def _():
            pltpu.sync_copy(ref.at[pl.ds(TAIL_BASE, TAIL_ROWS)],
                            acc.at[pl.ds(TAIL_BASE, TAIL_ROWS)])

    @pl.when(c == 0)
    def _():
        init_from(feats)

    @pl.when(c != 0)
    def _():
        init_from(zeros)

    row0 = w * rpw
    pltpu.sync_copy(srcs.at[pl.ds(row0, rpw)], idx_s)
    plsc.subcore_barrier()

    rows = (rows0, rows1)
    idx_d = (idx_d0, idx_d1)
    gsem = (gsem0, gsem1)
    ssem = (ssem0, ssem1)
    dsem = (dsem0, dsem1)

    def issue_gather(k, b):
        pltpu.async_copy(feats.at[idx_s.at[k]], rows[b], gsem[b])
        pltpu.async_copy(dsts.at[row0 + k], idx_d[b], dsem[b])

    def issue_scatter(b):
        pltpu.async_copy(rows[b], acc.at[idx_d[b]], ssem[b], add=True)

    def wait_gather(b):
        pltpu.make_async_copy(feats.at[idx_s.at[0]], rows[b], gsem[b]).wait()
        pltpu.make_async_copy(dsts.at[row0], idx_d[b], dsem[b]).wait()

    def wait_scatter(b):
        pltpu.make_async_copy(rows[b], acc.at[idx_d[b]], ssem[b]).wait()

    issue_gather(0, 0)

    def pair(g, carry):
        for b in (0, 1):
            k = 2 * g + b
            wait_gather(b)

            @pl.when(k + 1 < rpw)
            def _():
                @pl.when(k >= 1)
                def _():
                    wait_scatter(1 - b)

                issue_gather(k + 1, 1 - b)

            issue_scatter(b)
        return carry

    lax.fori_loop(0, rpw // 2, pair, 0)
    wait_scatter(0)
    wait_scatter(1)

    plsc.subcore_barrier()
    pltpu.sync_copy(acc.at[pl.ds(base, ROWS_PER_TILE)],
                    out.at[c, pl.ds(base, ROWS_PER_TILE)])

    @pl.when(s == N_SUB - 1)
    def _():
        pltpu.sync_copy(acc.at[pl.ds(TAIL_BASE, TAIL_ROWS)],
                        out.at[c, pl.ds(TAIL_BASE, TAIL_ROWS)])


def _sc_agg(feats, srcs, dsts, zeros):
    rpw = srcs.shape[0] // N_WORK
    fn = pl.kernel(
        _sc_agg_body,
        out_type=jax.ShapeDtypeStruct((N_CORES, N_NODES, D), jnp.float32),
        mesh=plsc.VectorSubcoreMesh(core_axis_name="c", subcore_axis_name="s",
                                    num_cores=N_CORES, num_subcores=N_SUB),
        scratch_types=[
            pltpu.VMEM((rpw, CHUNK), jnp.int32),
            pltpu.VMEM((CHUNK,), jnp.int32),
            pltpu.VMEM((CHUNK,), jnp.int32),
            pltpu.VMEM((CHUNK, D), jnp.float32),
            pltpu.VMEM((CHUNK, D), jnp.float32),
            pltpu.VMEM_SHARED((ACC_ROWS, D), jnp.float32),
            pltpu.SemaphoreType.DMA,
            pltpu.SemaphoreType.DMA,
            pltpu.SemaphoreType.DMA,
            pltpu.SemaphoreType.DMA,
            pltpu.SemaphoreType.DMA,
            pltpu.SemaphoreType.DMA,
        ],
    )
    return fn(feats, srcs, dsts, zeros)


ROW_BLK = 2000


def _mlp_body(p_ref, wa, ba, wb, bb, out_ref):
    h = p_ref[0] + p_ref[1]
    t = jnp.maximum(jnp.dot(h, wa[...], preferred_element_type=jnp.float32) + ba[...], 0.0)
    out_ref[...] = jnp.dot(t, wb[...], preferred_element_type=jnp.float32) + bb[...]


def _mlp(p, Wa, ba, Wb, bb):
    return pl.pallas_call(
        _mlp_body,
        grid=(N_NODES // ROW_BLK,),
        in_specs=[
            pl.BlockSpec((N_CORES, ROW_BLK, D), lambda i: (0, i, 0)),
            pl.BlockSpec((D, D), lambda i: (0, 0)),
            pl.BlockSpec((1, D), lambda i: (0, 0)),
            pl.BlockSpec((D, D), lambda i: (0, 0)),
            pl.BlockSpec((1, D), lambda i: (0, 0)),
        ],
        out_specs=pl.BlockSpec((ROW_BLK, D), lambda i: (i, 0)),
        out_shape=jax.ShapeDtypeStruct((N_NODES, D), jnp.float32),
    )(p, Wa, ba.reshape(1, D), Wb, bb.reshape(1, D))


def _mlp_pool_body(p_ref, wa, ba, wb, bb, wc, bcp, out_ref, acc):
    i = pl.program_id(0)

    @pl.when(i == 0)
    def _():
        acc[...] = jnp.zeros_like(acc)

    h = p_ref[0] + p_ref[1]
    t = jnp.maximum(jnp.dot(h, wa[...], preferred_element_type=jnp.float32) + ba[...], 0.0)
    h2 = jnp.dot(t, wb[...], preferred_element_type=jnp.float32) + bb[...]
    acc[...] += jnp.sum(h2, axis=0, keepdims=True)

    @pl.when(i == pl.num_programs(0) - 1)
    def _():
        out_ref[...] = jnp.dot(acc[...] * (1.0 / N_NODES), wc[...],
                               preferred_element_type=jnp.float32) + bcp[...]


def _mlp_pool(p, Wa, ba, Wb, bb, Wcp, bcp):
    return pl.pallas_call(
        _mlp_pool_body,
        grid=(N_NODES // ROW_BLK,),
        in_specs=[
            pl.BlockSpec((N_CORES, ROW_BLK, D), lambda i: (0, i, 0)),
            pl.BlockSpec((D, D), lambda i: (0, 0)),
            pl.BlockSpec((1, D), lambda i: (0, 0)),
            pl.BlockSpec((D, D), lambda i: (0, 0)),
            pl.BlockSpec((1, D), lambda i: (0, 0)),
            pl.BlockSpec((D, D), lambda i: (0, 0)),
            pl.BlockSpec((1, D), lambda i: (0, 0)),
        ],
        out_specs=pl.BlockSpec((1, D), lambda i: (0, 0)),
        out_shape=jax.ShapeDtypeStruct((1, D), jnp.float32),
        scratch_shapes=[pltpu.VMEM((1, D), jnp.float32)],
    )(p, Wa, ba.reshape(1, D), Wb, bb.reshape(1, D), Wcp, bcp)


def kernel(x, edge_index, W1a, b1a, W1b, b1b, W2a, b2a, W2b, b2b, Wc, bc):
    ei = edge_index.astype(jnp.int32)
    src, dst = ei[0], ei[1]
    n_edges = src.shape[0]
    rpw = -(-n_edges // (N_WORK * CHUNK))
    rpw = -(-rpw // 8) * 8
    pad = N_WORK * rpw * CHUNK - n_edges
    src_p = jnp.concatenate([src, jnp.zeros((pad,), jnp.int32)]).reshape(N_WORK * rpw, CHUNK)
    trash = N_NODES + jnp.arange(pad, dtype=jnp.int32) % TRASH_ROWS
    dst_p = jnp.concatenate([dst, trash]).reshape(N_WORK * rpw, CHUNK)
    zeros = jnp.zeros((N_NODES, D), jnp.float32)

    p1 = _sc_agg(x, src_p, dst_p, zeros)
    h1 = _mlp(p1, W1a, b1a, W1b, b1b)
    p2 = _sc_agg(h1, src_p, dst_p, zeros)

    n_cls = Wc.shape[1]
    Wcp = jnp.pad(Wc, ((0, 0), (0, D - n_cls)))
    bcp = jnp.pad(bc, (0, D - n_cls)).reshape(1, D)
    out = _mlp_pool(p2, W2a, b2a, W2b, b2b, Wcp, bcp)
    return out[:, :n_cls]


# speedup vs baseline: 2.4352x; 2.2590x over previous
import jax
import jax.numpy as jnp
from jax import lax
from jax.experimental import pallas as pl
from jax.experimental.pallas import tpu as pltpu
from jax.experimental.pallas import tpu_sc as plsc

N_NODES = 10000
D = 128
N_CORES = 2
N_SUB = 16
CHUNK = 128
SPLIT0 = 0.77
ROWS_PER_TILE = 624
TAIL_BASE = ROWS_PER_TILE * N_SUB
TAIL_ROWS = N_NODES - TAIL_BASE
TRASH_ROWS = 176
ACC_ROWS = N_NODES + TRASH_ROWS


def _sc_agg_body(feats, s0, d0, s1, d1, zeros, out, idx_s, idx_d0, idx_d1,
                 rows0, rows1, acc, gsem0, gsem1, ssem0, ssem1, dsem0, dsem1):
    c = lax.axis_index("c")
    s = lax.axis_index("s")

    base = s * ROWS_PER_TILE

    def init_from(ref):
        pltpu.sync_copy(ref.at[pl.ds(base, ROWS_PER_TILE)],
                        acc.at[pl.ds(base, ROWS_PER_TILE)])

        @pl.when(s == N_SUB - 1)
        def _():
            pltpu.sync_copy(ref.at[pl.ds(TAIL_BASE, TAIL_ROWS)],
                            acc.at[pl.ds(TAIL_BASE, TAIL_ROWS)])

    @pl.when(c == 0)
    def _():
        init_from(feats)

    @pl.when(c != 0)
    def _():
        init_from(zeros)

    plsc.subcore_barrier()

    rows = (rows0, rows1)
    idx_d = (idx_d0, idx_d1)
    gsem = (gsem0, gsem1)
    ssem = (ssem0, ssem1)
    dsem = (dsem0, dsem1)

    def run(srcs3, dsts3):
        k_chunks = srcs3.shape[1]
        pltpu.sync_copy(srcs3.at[s], idx_s.at[pl.ds(0, k_chunks)])

        def issue_gather(k, b):
            pltpu.async_copy(feats.at[idx_s.at[k]], rows[b], gsem[b])
            pltpu.async_copy(dsts3.at[s, k], idx_d[b], dsem[b])

        def issue_scatter(b):
            pltpu.async_copy(rows[b], acc.at[idx_d[b]], ssem[b], add=True)

        def wait_gather(b):
            pltpu.make_async_copy(feats.at[idx_s.at[0]], rows[b], gsem[b]).wait()
            pltpu.make_async_copy(dsts3.at[s, 0], idx_d[b], dsem[b]).wait()

        def wait_scatter(b):
            pltpu.make_async_copy(rows[b], acc.at[idx_d[b]], ssem[b]).wait()

        issue_gather(0, 0)

        def pair(g, carry):
            for b in (0, 1):
                k = 2 * g + b
                wait_gather(b)

                @pl.when(k + 1 < k_chunks)
                def _():
                    @pl.when(k >= 1)
                    def _():
                        wait_scatter(1 - b)

                    issue_gather(k + 1, 1 - b)

                issue_scatter(b)
            return carry

        lax.fori_loop(0, k_chunks // 2, pair, 0)
        if k_chunks % 2:
            k = k_chunks - 1
            b = k % 2
            wait_gather(b)
            issue_scatter(b)
        wait_scatter(0)
        wait_scatter(1)

    @pl.when(c == 0)
    def _():
        run(s0, d0)

    @pl.when(c != 0)
    def _():
        run(s1, d1)

    plsc.subcore_barrier()
    pltpu.sync_copy(acc.at[pl.ds(base, ROWS_PER_TILE)],
                    out.at[c, pl.ds(base, ROWS_PER_TILE)])

    @pl.when(s == N_SUB - 1)
    def _():
        pltpu.sync_copy(acc.at[pl.ds(TAIL_BASE, TAIL_ROWS)],
                        out.at[c, pl.ds(TAIL_BASE, TAIL_ROWS)])


def _sc_agg(feats, s0, d0, s1, d1, zeros):
    fn = pl.kernel(
        _sc_agg_body,
        out_type=jax.ShapeDtypeStruct((N_CORES, N_NODES, D), jnp.float32),
        mesh=plsc.VectorSubcoreMesh(core_axis_name="c", subcore_axis_name="s",
                                    num_cores=N_CORES, num_subcores=N_SUB),
        scratch_types=[
            pltpu.VMEM((s0.shape[1], CHUNK), jnp.int32),
            pltpu.VMEM((CHUNK,), jnp.int32),
            pltpu.VMEM((CHUNK,), jnp.int32),
            pltpu.VMEM((CHUNK, D), jnp.float32),
            pltpu.VMEM((CHUNK, D), jnp.float32),
            pltpu.VMEM_SHARED((ACC_ROWS, D), jnp.float32),
            pltpu.SemaphoreType.DMA,
            pltpu.SemaphoreType.DMA,
            pltpu.SemaphoreType.DMA,
            pltpu.SemaphoreType.DMA,
            pltpu.SemaphoreType.DMA,
            pltpu.SemaphoreType.DMA,
        ],
    )
    return fn(feats, s0, d0, s1, d1, zeros)


ROW_BLK = 2000


def _mlp_body(p_ref, wa, ba, wb, bb, out_ref):
    h = p_ref[0] + p_ref[1]
    t = jnp.maximum(jnp.dot(h, wa[...], preferred_element_type=jnp.float32) + ba[...], 0.0)
    out_ref[...] = jnp.dot(t, wb[...], preferred_element_type=jnp.float32) + bb[...]


def _mlp(p, Wa, ba, Wb, bb):
    return pl.pallas_call(
        _mlp_body,
        grid=(N_NODES // ROW_BLK,),
        in_specs=[
            pl.BlockSpec((N_CORES, ROW_BLK, D), lambda i: (0, i, 0)),
            pl.BlockSpec((D, D), lambda i: (0, 0)),
            pl.BlockSpec((1, D), lambda i: (0, 0)),
            pl.BlockSpec((D, D), lambda i: (0, 0)),
            pl.BlockSpec((1, D), lambda i: (0, 0)),
        ],
        out_specs=pl.BlockSpec((ROW_BLK, D), lambda i: (i, 0)),
        out_shape=jax.ShapeDtypeStruct((N_NODES, D), jnp.float32),
    )(p, Wa, ba.reshape(1, D), Wb, bb.reshape(1, D))


def _mlp_pool_body(p_ref, wa, ba, wb, bb, wc, bcp, out_ref, acc):
    i = pl.program_id(0)

    @pl.when(i == 0)
    def _():
        acc[...] = jnp.zeros_like(acc)

    h = p_ref[0] + p_ref[1]
    t = jnp.maximum(jnp.dot(h, wa[...], preferred_element_type=jnp.float32) + ba[...], 0.0)
    h2 = jnp.dot(t, wb[...], preferred_element_type=jnp.float32) + bb[...]
    acc[...] += jnp.sum(h2, axis=0, keepdims=True)

    @pl.when(i == pl.num_programs(0) - 1)
    def _():
        out_ref[...] = jnp.dot(acc[...] * (1.0 / N_NODES), wc[...],
                               preferred_element_type=jnp.float32) + bcp[...]


def _mlp_pool(p, Wa, ba, Wb, bb, Wcp, bcp):
    return pl.pallas_call(
        _mlp_pool_body,
        grid=(N_NODES // ROW_BLK,),
        in_specs=[
            pl.BlockSpec((N_CORES, ROW_BLK, D), lambda i: (0, i, 0)),
            pl.BlockSpec((D, D), lambda i: (0, 0)),
            pl.BlockSpec((1, D), lambda i: (0, 0)),
            pl.BlockSpec((D, D), lambda i: (0, 0)),
            pl.BlockSpec((1, D), lambda i: (0, 0)),
            pl.BlockSpec((D, D), lambda i: (0, 0)),
            pl.BlockSpec((1, D), lambda i: (0, 0)),
        ],
        out_specs=pl.BlockSpec((1, D), lambda i: (0, 0)),
        out_shape=jax.ShapeDtypeStruct((1, D), jnp.float32),
        scratch_shapes=[pltpu.VMEM((1, D), jnp.float32)],
    )(p, Wa, ba.reshape(1, D), Wb, bb.reshape(1, D), Wcp, bcp)


def kernel(x, edge_index, W1a, b1a, W1b, b1b, W2a, b2a, W2b, b2b, Wc, bc):
    ei = edge_index.astype(jnp.int32)
    src, dst = ei[0], ei[1]
    n_edges = src.shape[0]
    rt = -(-n_edges // (CHUNK * N_SUB))
    r0 = max(1, min(rt - 1, round(rt * SPLIT0)))
    r1 = rt - r0
    e0 = N_SUB * r0 * CHUNK
    pad = N_SUB * r1 * CHUNK - (n_edges - e0)
    trash = N_NODES + jnp.arange(pad, dtype=jnp.int32) % TRASH_ROWS
    s0 = src[:e0].reshape(N_SUB, r0, CHUNK)
    d0 = dst[:e0].reshape(N_SUB, r0, CHUNK)
    s1 = jnp.concatenate([src[e0:], jnp.zeros((pad,), jnp.int32)]).reshape(N_SUB, r1, CHUNK)
    d1 = jnp.concatenate([dst[e0:], trash]).reshape(N_SUB, r1, CHUNK)
    zeros = jnp.zeros((N_NODES, D), jnp.float32)

    p1 = _sc_agg(x, s0, d0, s1, d1, zeros)
    h1 = _mlp(p1, W1a, b1a, W1b, b1b)
    p2 = _sc_agg(h1, s0, d0, s1, d1, zeros)

    n_cls = Wc.shape[1]
    Wcp = jnp.pad(Wc, ((0, 0), (0, D - n_cls)))
    bcp = jnp.pad(bc, (0, D - n_cls)).reshape(1, D)
    out = _mlp_pool(p2, W2a, b2a, W2b, b2b, Wcp, bcp)
    return out[:, :n_cls]


# speedup vs baseline: 2.9053x; 1.1930x over previous
import jax
import jax.numpy as jnp
from jax import lax
from jax.experimental import pallas as pl
from jax.experimental.pallas import tpu as pltpu
from jax.experimental.pallas import tpu_sc as plsc

N_NODES = 10000
D = 128
N_CORES = 2
N_SUB = 16
CHUNK = 128
SPLIT0 = 0.71
ROWS_PER_TILE = 624
TAIL_BASE = ROWS_PER_TILE * N_SUB
TAIL_ROWS = N_NODES - TAIL_BASE
ACC_ROWS = N_NODES


def _sc_agg(feats, ei, zeros):
    n_edges = ei.shape[1]
    t_chunks = n_edges // CHUNK
    c0 = int(round(t_chunks * SPLIT0))
    c1 = t_chunks - c0
    k0max = -(-c0 // N_SUB)
    k1max = -(-c1 // N_SUB)

    def body(feats, ei, zeros, out, idx_s, idx_d0, idx_d1, rows0, rows1, acc,
             gsem0, gsem1, ssem0, ssem1, dsem0, dsem1):
        c = lax.axis_index("c")
        s = lax.axis_index("s")

        base = s * ROWS_PER_TILE

        def init_from(ref):
            pltpu.sync_copy(ref.at[pl.ds(base, ROWS_PER_TILE)],
                            acc.at[pl.ds(base, ROWS_PER_TILE)])

            @pl.when(s == N_SUB - 1)
            def _():
                pltpu.sync_copy(ref.at[pl.ds(TAIL_BASE, TAIL_ROWS)],
                                acc.at[pl.ds(TAIL_BASE, TAIL_ROWS)])

        @pl.when(c == 0)
        def _():
            init_from(feats)

        @pl.when(c != 0)
        def _():
            init_from(zeros)

        plsc.subcore_barrier()

        rows = (rows0, rows1)
        idx_d = (idx_d0, idx_d1)
        gsem = (gsem0, gsem1)
        ssem = (ssem0, ssem1)
        dsem = (dsem0, dsem1)

        def run(cbase, csize, kmax):
            r_lo = s * csize // N_SUB
            cnt = (s + 1) * csize // N_SUB - r_lo
            off = pl.multiple_of((cbase + r_lo) * CHUNK, CHUNK)

            pltpu.sync_copy(ei.at[0, pl.ds(off, kmax * CHUNK)],
                            idx_s.at[pl.ds(0, kmax * CHUNK)])

            def issue_gather(k, b):
                kof = pl.multiple_of(k * CHUNK, CHUNK)
                pltpu.async_copy(feats.at[idx_s.at[pl.ds(kof, CHUNK)]],
                                 rows[b], gsem[b])
                pltpu.async_copy(ei.at[1, pl.ds(off + kof, CHUNK)],
                                 idx_d[b], dsem[b])

            def issue_scatter(b):
                pltpu.async_copy(rows[b], acc.at[idx_d[b]], ssem[b], add=True)

            def wait_gather(b):
                pltpu.make_async_copy(feats.at[idx_s.at[pl.ds(0, CHUNK)]],
                                      rows[b], gsem[b]).wait()
                pltpu.make_async_copy(ei.at[1, pl.ds(0, CHUNK)],
                                      idx_d[b], dsem[b]).wait()

            def wait_scatter(b):
                pltpu.make_async_copy(rows[b], acc.at[idx_d[b]], ssem[b]).wait()

            issue_gather(0, 0)

            def pair(g, carry):
                for b in (0, 1):
                    k = 2 * g + b
                    wait_gather(b)

                    @pl.when(k + 1 < cnt)
                    def _():
                        @pl.when(k >= 1)
                        def _():
                            wait_scatter(1 - b)

                        issue_gather(k + 1, 1 - b)

                    issue_scatter(b)
                return carry

            lax.fori_loop(0, cnt // 2, pair, 0)

            @pl.when(cnt % 2 == 1)
            def _():
                wait_gather(0)
                issue_scatter(0)

            wait_scatter(0)
            wait_scatter(1)

        @pl.when(c == 0)
        def _():
            run(0, c0, k0max)

        @pl.when(c != 0)
        def _():
            run(c0, c1, k1max)

        plsc.subcore_barrier()
        pltpu.sync_copy(acc.at[pl.ds(base, ROWS_PER_TILE)],
                        out.at[c, pl.ds(base, ROWS_PER_TILE)])

        @pl.when(s == N_SUB - 1)
        def _():
            pltpu.sync_copy(acc.at[pl.ds(TAIL_BASE, TAIL_ROWS)],
                            out.at[c, pl.ds(TAIL_BASE, TAIL_ROWS)])

    fn = pl.kernel(
        body,
        out_type=jax.ShapeDtypeStruct((N_CORES, N_NODES, D), jnp.float32),
        mesh=plsc.VectorSubcoreMesh(core_axis_name="c", subcore_axis_name="s",
                                    num_cores=N_CORES, num_subcores=N_SUB),
        scratch_types=[
            pltpu.VMEM((max(k0max, k1max) * CHUNK,), jnp.int32),
            pltpu.VMEM((CHUNK,), jnp.int32),
            pltpu.VMEM((CHUNK,), jnp.int32),
            pltpu.VMEM((CHUNK, D), jnp.float32),
            pltpu.VMEM((CHUNK, D), jnp.float32),
            pltpu.VMEM_SHARED((ACC_ROWS, D), jnp.float32),
            pltpu.SemaphoreType.DMA,
            pltpu.SemaphoreType.DMA,
            pltpu.SemaphoreType.DMA,
            pltpu.SemaphoreType.DMA,
            pltpu.SemaphoreType.DMA,
            pltpu.SemaphoreType.DMA,
        ],
    )
    return fn(feats, ei, zeros)


ROW_BLK = 2000


def _mlp_body(p_ref, wa, ba, wb, bb, out_ref):
    h = p_ref[0] + p_ref[1]
    t = jnp.maximum(jnp.dot(h, wa[...], preferred_element_type=jnp.float32) + ba[...], 0.0)
    out_ref[...] = jnp.dot(t, wb[...], preferred_element_type=jnp.float32) + bb[...]


def _mlp(p, Wa, ba, Wb, bb):
    return pl.pallas_call(
        _mlp_body,
        grid=(N_NODES // ROW_BLK,),
        in_specs=[
            pl.BlockSpec((N_CORES, ROW_BLK, D), lambda i: (0, i, 0)),
            pl.BlockSpec((D, D), lambda i: (0, 0)),
            pl.BlockSpec((1, D), lambda i: (0, 0)),
            pl.BlockSpec((D, D), lambda i: (0, 0)),
            pl.BlockSpec((1, D), lambda i: (0, 0)),
        ],
        out_specs=pl.BlockSpec((ROW_BLK, D), lambda i: (i, 0)),
        out_shape=jax.ShapeDtypeStruct((N_NODES, D), jnp.float32),
    )(p, Wa, ba.reshape(1, D), Wb, bb.reshape(1, D))


def _mlp_pool_body(p_ref, wa, ba, wb, bb, wc, bcp, out_ref, acc):
    i = pl.program_id(0)

    @pl.when(i == 0)
    def _():
        acc[...] = jnp.zeros_like(acc)

    h = p_ref[0] + p_ref[1]
    t = jnp.maximum(jnp.dot(h, wa[...], preferred_element_type=jnp.float32) + ba[...], 0.0)
    h2 = jnp.dot(t, wb[...], preferred_element_type=jnp.float32) + bb[...]
    acc[...] += jnp.sum(h2, axis=0, keepdims=True)

    @pl.when(i == pl.num_programs(0) - 1)
    def _():
        out_ref[...] = jnp.dot(acc[...] * (1.0 / N_NODES), wc[...],
                               preferred_element_type=jnp.float32) + bcp[...]


def _mlp_pool(p, Wa, ba, Wb, bb, Wcp, bcp):
    return pl.pallas_call(
        _mlp_pool_body,
        grid=(N_NODES // ROW_BLK,),
        in_specs=[
            pl.BlockSpec((N_CORES, ROW_BLK, D), lambda i: (0, i, 0)),
            pl.BlockSpec((D, D), lambda i: (0, 0)),
            pl.BlockSpec((1, D), lambda i: (0, 0)),
            pl.BlockSpec((D, D), lambda i: (0, 0)),
            pl.BlockSpec((1, D), lambda i: (0, 0)),
            pl.BlockSpec((D, D), lambda i: (0, 0)),
            pl.BlockSpec((1, D), lambda i: (0, 0)),
        ],
        out_specs=pl.BlockSpec((1, D), lambda i: (0, 0)),
        out_shape=jax.ShapeDtypeStruct((1, D), jnp.float32),
        scratch_shapes=[pltpu.VMEM((1, D), jnp.float32)],
    )(p, Wa, ba.reshape(1, D), Wb, bb.reshape(1, D), Wcp, bcp)


def kernel(x, edge_index, W1a, b1a, W1b, b1b, W2a, b2a, W2b, b2b, Wc, bc):
    ei = edge_index.astype(jnp.int32)
    assert ei.shape[1] % CHUNK == 0
    zeros = jnp.zeros((N_NODES, D), jnp.float32)

    p1 = _sc_agg(x, ei, zeros)
    h1 = _mlp(p1, W1a, b1a, W1b, b1b)
    p2 = _sc_agg(h1, ei, zeros)

    n_cls = Wc.shape[1]
    Wcp = jnp.pad(Wc, ((0, 0), (0, D - n_cls)))
    bcp = jnp.pad(bc, (0, D - n_cls)).reshape(1, D)
    out = _mlp_pool(p2, W2a, b2a, W2b, b2b, Wcp, bcp)
    return out[:, :n_cls]


# speedup vs baseline: 3.5735x; 1.2300x over previous
import jax
import jax.numpy as jnp
from jax import lax
from jax.experimental import pallas as pl
from jax.experimental.pallas import tpu as pltpu
from jax.experimental.pallas import tpu_sc as plsc

N_NODES = 10000
D = 128
N_CORES = 2
N_SUB = 16
CHUNK = 128
SPLIT0 = 0.54
ROWS_PER_TILE = 624
TAIL_BASE = ROWS_PER_TILE * N_SUB
TAIL_ROWS = N_NODES - TAIL_BASE
ACC_ROWS = N_NODES


def _sc_agg(feats, ei, zeros):
    n_edges = ei.shape[1]
    t_chunks = n_edges // CHUNK
    c0 = int(round(t_chunks * SPLIT0))
    c1 = t_chunks - c0
    k0max = -(-c0 // N_SUB)
    k1max = -(-c1 // N_SUB)

    def body(feats, ei, zeros, out, idx_s, idx_d0, idx_d1, rows0, rows1, acc,
             gsem0, gsem1, ssem0, ssem1, dsem0, dsem1):
        c = lax.axis_index("c")
        s = lax.axis_index("s")

        base = s * ROWS_PER_TILE

        def init_from(ref):
            pltpu.sync_copy(ref.at[pl.ds(base, ROWS_PER_TILE)],
                            acc.at[pl.ds(base, ROWS_PER_TILE)])

            @pl.when(s == N_SUB - 1)
            def _():
                pltpu.sync_copy(ref.at[pl.ds(TAIL_BASE, TAIL_ROWS)],
                                acc.at[pl.ds(TAIL_BASE, TAIL_ROWS)])

        @pl.when(c == 0)
        def _():
            init_from(feats)

        @pl.when(c != 0)
        def _():
            init_from(zeros)

        plsc.subcore_barrier()

        rows = (rows0, rows1)
        idx_d = (idx_d0, idx_d1)
        gsem = (gsem0, gsem1)
        ssem = (ssem0, ssem1)
        dsem = (dsem0, dsem1)

        def run(cbase, csize, kmax):
            r_lo = s * csize // N_SUB
            cnt = (s + 1) * csize // N_SUB - r_lo
            off = pl.multiple_of((cbase + r_lo) * CHUNK, CHUNK)

            pltpu.sync_copy(ei.at[0, pl.ds(off, kmax * CHUNK)],
                            idx_s.at[pl.ds(0, kmax * CHUNK)])

            def issue_gather(k, b):
                kof = pl.multiple_of(k * CHUNK, CHUNK)
                pltpu.async_copy(feats.at[idx_s.at[pl.ds(kof, CHUNK)]],
                                 rows[b], gsem[b])
                pltpu.async_copy(ei.at[1, pl.ds(off + kof, CHUNK)],
                                 idx_d[b], dsem[b])

            def issue_scatter(b):
                pltpu.async_copy(rows[b], acc.at[idx_d[b]], ssem[b], add=True)

            def wait_gather(b):
                pltpu.make_async_copy(feats.at[idx_s.at[pl.ds(0, CHUNK)]],
                                      rows[b], gsem[b]).wait()
                pltpu.make_async_copy(ei.at[1, pl.ds(0, CHUNK)],
                                      idx_d[b], dsem[b]).wait()

            def wait_scatter(b):
                pltpu.make_async_copy(rows[b], acc.at[idx_d[b]], ssem[b]).wait()

            issue_gather(0, 0)

            def pair(g, carry):
                for b in (0, 1):
                    k = 2 * g + b
                    wait_gather(b)

                    @pl.when(k + 1 < cnt)
                    def _():
                        @pl.when(k >= 1)
                        def _():
                            wait_scatter(1 - b)

                        issue_gather(k + 1, 1 - b)

                    issue_scatter(b)
                return carry

            lax.fori_loop(0, cnt // 2, pair, 0)

            @pl.when(cnt % 2 == 1)
            def _():
                wait_gather(0)
                issue_scatter(0)

            wait_scatter(0)
            wait_scatter(1)

        @pl.when(c == 0)
        def _():
            run(0, c0, k0max)

        @pl.when(c != 0)
        def _():
            run(c0, c1, k1max)

        plsc.subcore_barrier()
        pltpu.sync_copy(acc.at[pl.ds(base, ROWS_PER_TILE)],
                        out.at[c, pl.ds(base, ROWS_PER_TILE)])

        @pl.when(s == N_SUB - 1)
        def _():
            pltpu.sync_copy(acc.at[pl.ds(TAIL_BASE, TAIL_ROWS)],
                            out.at[c, pl.ds(TAIL_BASE, TAIL_ROWS)])

    fn = pl.kernel(
        body,
        out_type=jax.ShapeDtypeStruct((N_CORES, N_NODES, D), jnp.float32),
        mesh=plsc.VectorSubcoreMesh(core_axis_name="c", subcore_axis_name="s",
                                    num_cores=N_CORES, num_subcores=N_SUB),
        scratch_types=[
            pltpu.VMEM((max(k0max, k1max) * CHUNK,), jnp.int32),
            pltpu.VMEM((CHUNK,), jnp.int32),
            pltpu.VMEM((CHUNK,), jnp.int32),
            pltpu.VMEM((CHUNK, D), jnp.float32),
            pltpu.VMEM((CHUNK, D), jnp.float32),
            pltpu.VMEM_SHARED((ACC_ROWS, D), jnp.float32),
            pltpu.SemaphoreType.DMA,
            pltpu.SemaphoreType.DMA,
            pltpu.SemaphoreType.DMA,
            pltpu.SemaphoreType.DMA,
            pltpu.SemaphoreType.DMA,
            pltpu.SemaphoreType.DMA,
        ],
    )
    return fn(feats, ei, zeros)


ROW_BLK = 2000


def _mlp_body(p_ref, wa, ba, wb, bb, out_ref):
    h = p_ref[0] + p_ref[1]
    t = jnp.maximum(jnp.dot(h, wa[...], preferred_element_type=jnp.float32) + ba[...], 0.0)
    out_ref[...] = jnp.dot(t, wb[...], preferred_element_type=jnp.float32) + bb[...]


def _mlp(p, Wa, ba, Wb, bb):
    return pl.pallas_call(
        _mlp_body,
        grid=(N_NODES // ROW_BLK,),
        in_specs=[
            pl.BlockSpec((N_CORES, ROW_BLK, D), lambda i: (0, i, 0)),
            pl.BlockSpec((D, D), lambda i: (0, 0)),
            pl.BlockSpec((1, D), lambda i: (0, 0)),
            pl.BlockSpec((D, D), lambda i: (0, 0)),
            pl.BlockSpec((1, D), lambda i: (0, 0)),
        ],
        out_specs=pl.BlockSpec((ROW_BLK, D), lambda i: (i, 0)),
        out_shape=jax.ShapeDtypeStruct((N_NODES, D), jnp.float32),
    )(p, Wa, ba.reshape(1, D), Wb, bb.reshape(1, D))


def _mlp_pool_body(p_ref, wa, ba, wb, bb, wc, bcp, out_ref, acc):
    i = pl.program_id(0)

    @pl.when(i == 0)
    def _():
        acc[...] = jnp.zeros_like(acc)

    h = p_ref[0] + p_ref[1]
    t = jnp.maximum(jnp.dot(h, wa[...], preferred_element_type=jnp.float32) + ba[...], 0.0)
    h2 = jnp.dot(t, wb[...], preferred_element_type=jnp.float32) + bb[...]
    acc[...] += jnp.sum(h2, axis=0, keepdims=True)

    @pl.when(i == pl.num_programs(0) - 1)
    def _():
        out_ref[...] = jnp.dot(acc[...] * (1.0 / N_NODES), wc[...],
                               preferred_element_type=jnp.float32) + bcp[...]


def _mlp_pool(p, Wa, ba, Wb, bb, Wcp, bcp):
    return pl.pallas_call(
        _mlp_pool_body,
        grid=(N_NODES // ROW_BLK,),
        in_specs=[
            pl.BlockSpec((N_CORES, ROW_BLK, D), lambda i: (0, i, 0)),
            pl.BlockSpec((D, D), lambda i: (0, 0)),
            pl.BlockSpec((1, D), lambda i: (0, 0)),
            pl.BlockSpec((D, D), lambda i: (0, 0)),
            pl.BlockSpec((1, D), lambda i: (0, 0)),
            pl.BlockSpec((D, D), lambda i: (0, 0)),
            pl.BlockSpec((1, D), lambda i: (0, 0)),
        ],
        out_specs=pl.BlockSpec((1, D), lambda i: (0, 0)),
        out_shape=jax.ShapeDtypeStruct((1, D), jnp.float32),
        scratch_shapes=[pltpu.VMEM((1, D), jnp.float32)],
    )(p, Wa, ba.reshape(1, D), Wb, bb.reshape(1, D), Wcp, bcp)


def kernel(x, edge_index, W1a, b1a, W1b, b1b, W2a, b2a, W2b, b2b, Wc, bc):
    ei = edge_index.astype(jnp.int32)
    assert ei.shape[1] % CHUNK == 0
    zeros = jnp.zeros((N_NODES, D), jnp.float32)

    p1 = _sc_agg(x, ei, zeros)
    h1 = _mlp(p1, W1a, b1a, W1b, b1b)
    p2 = _sc_agg(h1, ei, zeros)

    n_cls = Wc.shape[1]
    Wcp = jnp.pad(Wc, ((0, 0), (0, D - n_cls)))
    bcp = jnp.pad(bc, (0, D - n_cls)).reshape(1, D)
    out = _mlp_pool(p2, W2a, b2a, W2b, b2b, Wcp, bcp)
    return out[:, :n_cls]


# speedup vs baseline: 3.7613x; 1.0525x over previous
import jax
import jax.numpy as jnp
from jax import lax
from jax.experimental import pallas as pl
from jax.experimental.pallas import tpu as pltpu
from jax.experimental.pallas import tpu_sc as plsc

N_NODES = 10000
D = 128
N_CORES = 2
N_SUB = 16
CHUNK = 128
SPLIT0 = 0.505
ROWS_PER_TILE = 624
TAIL_BASE = ROWS_PER_TILE * N_SUB
TAIL_ROWS = N_NODES - TAIL_BASE
ACC_ROWS = N_NODES


def _sc_agg(feats, ei, zeros):
    n_edges = ei.shape[1]
    t_chunks = n_edges // CHUNK
    c0 = int(round(t_chunks * SPLIT0))
    c1 = t_chunks - c0
    k0max = -(-c0 // N_SUB)
    k1max = -(-c1 // N_SUB)

    def body(feats, ei, zeros, out, idx_s, idx_d0, idx_d1, rows0, rows1, acc,
             gsem0, gsem1, ssem0, ssem1, dsem0, dsem1):
        c = lax.axis_index("c")
        s = lax.axis_index("s")

        base = s * ROWS_PER_TILE

        @pl.when(c == 0)
        def _():
            pltpu.sync_copy(feats.at[pl.ds(base, ROWS_PER_TILE)],
                            acc.at[pl.ds(base, ROWS_PER_TILE)])

            @pl.when(s == N_SUB - 1)
            def _():
                pltpu.sync_copy(feats.at[pl.ds(TAIL_BASE, TAIL_ROWS)],
                                acc.at[pl.ds(TAIL_BASE, TAIL_ROWS)])

        @pl.when(c != 0)
        def _():
            for j in range(ROWS_PER_TILE // CHUNK):
                pltpu.sync_copy(zeros,
                                acc.at[pl.ds(base + j * CHUNK, CHUNK)])
            rem = ROWS_PER_TILE % CHUNK
            pltpu.sync_copy(zeros.at[pl.ds(0, rem)],
                            acc.at[pl.ds(base + ROWS_PER_TILE - rem, rem)])

            @pl.when(s == N_SUB - 1)
            def _():
                pltpu.sync_copy(zeros.at[pl.ds(0, TAIL_ROWS)],
                                acc.at[pl.ds(TAIL_BASE, TAIL_ROWS)])

        plsc.subcore_barrier()

        rows = (rows0, rows1)
        idx_d = (idx_d0, idx_d1)
        gsem = (gsem0, gsem1)
        ssem = (ssem0, ssem1)
        dsem = (dsem0, dsem1)

        def run(cbase, csize, kmax):
            r_lo = s * csize // N_SUB
            cnt = (s + 1) * csize // N_SUB - r_lo
            off = pl.multiple_of((cbase + r_lo) * CHUNK, CHUNK)

            pltpu.sync_copy(ei.at[0, pl.ds(off, kmax * CHUNK)],
                            idx_s.at[pl.ds(0, kmax * CHUNK)])

            def issue_gather(k, b):
                kof = pl.multiple_of(k * CHUNK, CHUNK)
                pltpu.async_copy(feats.at[idx_s.at[pl.ds(kof, CHUNK)]],
                                 rows[b], gsem[b])
                pltpu.async_copy(ei.at[1, pl.ds(off + kof, CHUNK)],
                                 idx_d[b], dsem[b])

            def issue_scatter(b):
                pltpu.async_copy(rows[b], acc.at[idx_d[b]], ssem[b], add=True)

            def wait_gather(b):
                pltpu.make_async_copy(feats.at[idx_s.at[pl.ds(0, CHUNK)]],
                                      rows[b], gsem[b]).wait()
                pltpu.make_async_copy(ei.at[1, pl.ds(0, CHUNK)],
                                      idx_d[b], dsem[b]).wait()

            def wait_scatter(b):
                pltpu.make_async_copy(rows[b], acc.at[idx_d[b]], ssem[b]).wait()

            issue_gather(0, 0)

            def pair(g, carry):
                for b in (0, 1):
                    k = 2 * g + b
                    wait_gather(b)

                    @pl.when(k + 1 < cnt)
                    def _():
                        @pl.when(k >= 1)
                        def _():
                            wait_scatter(1 - b)

                        issue_gather(k + 1, 1 - b)

                    issue_scatter(b)
                return carry

            lax.fori_loop(0, cnt // 2, pair, 0)

            @pl.when(cnt % 2 == 1)
            def _():
                wait_gather(0)
                issue_scatter(0)

            wait_scatter(0)
            wait_scatter(1)

        @pl.when(c == 0)
        def _():
            run(0, c0, k0max)

        @pl.when(c != 0)
        def _():
            run(c0, c1, k1max)

        plsc.subcore_barrier()
        pltpu.sync_copy(acc.at[pl.ds(base, ROWS_PER_TILE)],
                        out.at[c, pl.ds(base, ROWS_PER_TILE)])

        @pl.when(s == N_SUB - 1)
        def _():
            pltpu.sync_copy(acc.at[pl.ds(TAIL_BASE, TAIL_ROWS)],
                            out.at[c, pl.ds(TAIL_BASE, TAIL_ROWS)])

    fn = pl.kernel(
        body,
        out_type=jax.ShapeDtypeStruct((N_CORES, N_NODES, D), jnp.float32),
        mesh=plsc.VectorSubcoreMesh(core_axis_name="c", subcore_axis_name="s",
                                    num_cores=N_CORES, num_subcores=N_SUB),
        scratch_types=[
            pltpu.VMEM((max(k0max, k1max) * CHUNK,), jnp.int32),
            pltpu.VMEM((CHUNK,), jnp.int32),
            pltpu.VMEM((CHUNK,), jnp.int32),
            pltpu.VMEM((CHUNK, D), jnp.float32),
            pltpu.VMEM((CHUNK, D), jnp.float32),
            pltpu.VMEM_SHARED((ACC_ROWS, D), jnp.float32),
            pltpu.SemaphoreType.DMA,
            pltpu.SemaphoreType.DMA,
            pltpu.SemaphoreType.DMA,
            pltpu.SemaphoreType.DMA,
            pltpu.SemaphoreType.DMA,
            pltpu.SemaphoreType.DMA,
        ],
    )
    return fn(feats, ei, zeros)


ROW_BLK = 2000


def _mlp_body(p_ref, wa, ba, wb, bb, out_ref):
    h = p_ref[0] + p_ref[1]
    t = jnp.maximum(jnp.dot(h, wa[...], preferred_element_type=jnp.float32) + ba[...], 0.0)
    out_ref[...] = jnp.dot(t, wb[...], preferred_element_type=jnp.float32) + bb[...]


def _mlp(p, Wa, ba, Wb, bb):
    return pl.pallas_call(
        _mlp_body,
        grid=(N_NODES // ROW_BLK,),
        in_specs=[
            pl.BlockSpec((N_CORES, ROW_BLK, D), lambda i: (0, i, 0)),
            pl.BlockSpec((D, D), lambda i: (0, 0)),
            pl.BlockSpec((1, D), lambda i: (0, 0)),
            pl.BlockSpec((D, D), lambda i: (0, 0)),
            pl.BlockSpec((1, D), lambda i: (0, 0)),
        ],
        out_specs=pl.BlockSpec((ROW_BLK, D), lambda i: (i, 0)),
        out_shape=jax.ShapeDtypeStruct((N_NODES, D), jnp.float32),
    )(p, Wa, ba.reshape(1, D), Wb, bb.reshape(1, D))


def _mlp_pool_body(p_ref, wa, ba, wb, bb, wc, bcp, out_ref, acc):
    i = pl.program_id(0)

    @pl.when(i == 0)
    def _():
        acc[...] = jnp.zeros_like(acc)

    h = p_ref[0] + p_ref[1]
    t = jnp.maximum(jnp.dot(h, wa[...], preferred_element_type=jnp.float32) + ba[...], 0.0)
    h2 = jnp.dot(t, wb[...], preferred_element_type=jnp.float32) + bb[...]
    acc[...] += jnp.sum(h2, axis=0, keepdims=True)

    @pl.when(i == pl.num_programs(0) - 1)
    def _():
        out_ref[...] = jnp.dot(acc[...] * (1.0 / N_NODES), wc[...],
                               preferred_element_type=jnp.float32) + bcp[...]


def _mlp_pool(p, Wa, ba, Wb, bb, Wcp, bcp):
    return pl.pallas_call(
        _mlp_pool_body,
        grid=(N_NODES // ROW_BLK,),
        in_specs=[
            pl.BlockSpec((N_CORES, ROW_BLK, D), lambda i: (0, i, 0)),
            pl.BlockSpec((D, D), lambda i: (0, 0)),
            pl.BlockSpec((1, D), lambda i: (0, 0)),
            pl.BlockSpec((D, D), lambda i: (0, 0)),
            pl.BlockSpec((1, D), lambda i: (0, 0)),
            pl.BlockSpec((D, D), lambda i: (0, 0)),
            pl.BlockSpec((1, D), lambda i: (0, 0)),
        ],
        out_specs=pl.BlockSpec((1, D), lambda i: (0, 0)),
        out_shape=jax.ShapeDtypeStruct((1, D), jnp.float32),
        scratch_shapes=[pltpu.VMEM((1, D), jnp.float32)],
    )(p, Wa, ba.reshape(1, D), Wb, bb.reshape(1, D), Wcp, bcp)


def kernel(x, edge_index, W1a, b1a, W1b, b1b, W2a, b2a, W2b, b2b, Wc, bc):
    ei = edge_index.astype(jnp.int32)
    assert ei.shape[1] % CHUNK == 0
    zeros = jnp.zeros((CHUNK, D), jnp.float32)

    p1 = _sc_agg(x, ei, zeros)
    h1 = _mlp(p1, W1a, b1a, W1b, b1b)
    p2 = _sc_agg(h1, ei, zeros)

    n_cls = Wc.shape[1]
    Wcp = jnp.pad(Wc, ((0, 0), (0, D - n_cls)))
    bcp = jnp.pad(bc, (0, D - n_cls)).reshape(1, D)
    out = _mlp_pool(p2, W2a, b2a, W2b, b2b, Wcp, bcp)
    return out[:, :n_cls]
